# baseline (speedup 1.0000x reference)
import functools

import jax
import jax.numpy as jnp
from jax.experimental import pallas as pl


def _matmul(x, w):
    M, K = x.shape
    _, Nc = w.shape
    BM = 400

    def mm(x_ref, w_ref, o_ref):
        o_ref[...] = jnp.dot(x_ref[...], w_ref[...],
                             preferred_element_type=jnp.float32)

    return pl.pallas_call(
        mm,
        grid=(M // BM,),
        in_specs=[pl.BlockSpec((BM, K), lambda i: (i, 0)),
                  pl.BlockSpec((K, Nc), lambda i: (0, 0))],
        out_specs=pl.BlockSpec((BM, Nc), lambda i: (i, 0)),
        out_shape=jax.ShapeDtypeStruct((M, Nc), jnp.float32),
    )(x, w)


def _gat_edge(h, src, dst, a_src, a_dst, b, n):
    alpha_src = h @ a_src
    alpha_dst = h @ a_dst
    e = alpha_src[src] + alpha_dst[dst]
    e = jax.nn.leaky_relu(e, negative_slope=0.2)
    e_max = jax.ops.segment_max(e, dst, num_segments=n)
    e_exp = jnp.exp(e - e_max[dst])
    denom = jax.ops.segment_sum(e_exp, dst, num_segments=n)
    alpha = e_exp / (denom[dst] + 1e-16)
    out = jax.ops.segment_sum(h[src] * alpha[:, None], dst, num_segments=n)
    return out + b


def kernel(x, edge_index, W1, a_src1, a_dst1, b1, W2, a_src2, a_dst2, b2,
           W3, a_src3, a_dst3, b3, W4, b4):
    n = x.shape[0]
    loop = jnp.arange(n, dtype=edge_index.dtype)
    src = jnp.concatenate([edge_index[0], loop])
    dst = jnp.concatenate([edge_index[1], loop])

    h = _matmul(x, W1)
    h = jax.nn.relu(_gat_edge(h, src, dst, a_src1, a_dst1, b1, n))
    h = _matmul(h, W2)
    h = jax.nn.relu(_gat_edge(h, src, dst, a_src2, a_dst2, b2, n))
    h = _matmul(h, W3)
    h = jax.nn.relu(_gat_edge(h, src, dst, a_src3, a_dst3, b3, n))

    ones = jnp.ones(src.shape[0], dtype=x.dtype)
    deg = jax.ops.segment_sum(ones, dst, num_segments=n)
    dinv = jnp.where(deg > 0, deg ** -0.5, 0.0)
    norm = dinv[src] * dinv[dst]
    h4 = _matmul(h, W4)
    z = jax.ops.segment_sum(h4[src] * norm[:, None], dst, num_segments=n)
    return z + b4


# speedup vs baseline: 7.6948x; 7.6948x over previous
import functools

import jax
import jax.numpy as jnp
from jax import lax
from jax.experimental import pallas as pl
from jax.experimental.pallas import tpu as pltpu
from jax.experimental.pallas import tpu_sc as plsc

_NT = 16
_NC = 2
_CK = 128
_JUNK = 112
_BM = 400


def _leaky(x):
    return jnp.maximum(x, 0.2 * x)


def _proj(x, W, a2):
    n, K = x.shape
    H = W.shape[1]
    NB = H // 128
    with_sd = a2 is not None

    def body(x_ref, w_ref, *rest):
        if with_sd:
            a_ref, hb_ref, sd_ref = rest
        else:
            (hb_ref,) = rest
        h = jnp.dot(x_ref[...], w_ref[...], preferred_element_type=jnp.float32)
        for b in range(NB):
            hb_ref[b, :, :] = h[:, b * 128:(b + 1) * 128]
        if with_sd:
            sd_ref[...] = jnp.dot(h, a_ref[...],
                                  preferred_element_type=jnp.float32)

    in_specs = [pl.BlockSpec((_BM, K), lambda i: (i, 0)),
                pl.BlockSpec((K, H), lambda i: (0, 0))]
    out_specs = [pl.BlockSpec((NB, _BM, 128), lambda i: (0, i, 0))]
    out_shape = [jax.ShapeDtypeStruct((NB, n, 128), jnp.float32)]
    args = [x, W]
    if with_sd:
        in_specs.append(pl.BlockSpec((H, 2), lambda i: (0, 0)))
        out_specs.append(pl.BlockSpec((_BM, 2), lambda i: (i, 0)))
        out_shape.append(jax.ShapeDtypeStruct((n, 2), jnp.float32))
        args.append(a2)
    res = pl.pallas_call(
        body, grid=(n // _BM,), in_specs=in_specs, out_specs=out_specs,
        out_shape=out_shape)(*args)
    return (res[0], res[1]) if with_sd else (res[0], None)


def _softmax_prep(sd):
    n = sd.shape[0]

    def body(sd_ref, o_ref):
        asrc = sd_ref[:, 0:1]
        adst = sd_ref[:, 1:2]
        m = jnp.max(asrc)
        cdst = _leaky(m + adst)
        pself = jnp.exp(_leaky(asrc + adst) - cdst)
        o_ref[...] = jnp.concatenate([cdst, pself], axis=1)

    return pl.pallas_call(
        body, grid=(1,),
        in_specs=[pl.BlockSpec((n, 2), lambda i: (0, 0))],
        out_specs=pl.BlockSpec((n, 2), lambda i: (0, 0)),
        out_shape=jax.ShapeDtypeStruct((n, 2), jnp.float32))(sd)


def _combine_gat(acc, hb, den, pself, bias):
    NB, n, _ = acc.shape
    H = NB * 128

    def body(acc_ref, hb_ref, den_ref, ps_ref, b_ref, o_ref):
        a = jnp.concatenate([acc_ref[b] for b in range(NB)], axis=1)
        h = jnp.concatenate([hb_ref[b] for b in range(NB)], axis=1)
        ps = ps_ref[...]
        dtot = den_ref[...] + ps
        o_ref[...] = jax.nn.relu((a + ps * h) / dtot + b_ref[...])

    return pl.pallas_call(
        body, grid=(n // _BM,),
        in_specs=[pl.BlockSpec((NB, _BM, 128), lambda i: (0, i, 0)),
                  pl.BlockSpec((NB, _BM, 128), lambda i: (0, i, 0)),
                  pl.BlockSpec((_BM, 1), lambda i: (i, 0)),
                  pl.BlockSpec((_BM, 1), lambda i: (i, 0)),
                  pl.BlockSpec((1, H), lambda i: (0, 0))],
        out_specs=pl.BlockSpec((_BM, H), lambda i: (i, 0)),
        out_shape=jax.ShapeDtypeStruct((n, H), jnp.float32),
    )(acc, hb, den, pself, bias)


def _dinv_kernel(deg):
    n = deg.shape[0]

    def body(d_ref, o_ref):
        o_ref[...] = lax.rsqrt(d_ref[...] + 1.0)

    return pl.pallas_call(
        body, grid=(1,),
        in_specs=[pl.BlockSpec((n, 1), lambda i: (0, 0))],
        out_specs=pl.BlockSpec((n, 1), lambda i: (0, 0)),
        out_shape=jax.ShapeDtypeStruct((n, 1), jnp.float32))(deg)


def _combine_gcn(acc, hb, dinv, bias):
    NB, n, _ = acc.shape
    H = NB * 128

    def body(acc_ref, hb_ref, di_ref, b_ref, o_ref):
        a = jnp.concatenate([acc_ref[b] for b in range(NB)], axis=1)
        h = jnp.concatenate([hb_ref[b] for b in range(NB)], axis=1)
        di = di_ref[...]
        o_ref[...] = a + (di * di) * h + b_ref[...]

    return pl.pallas_call(
        body, grid=(n // _BM,),
        in_specs=[pl.BlockSpec((NB, _BM, 128), lambda i: (0, i, 0)),
                  pl.BlockSpec((NB, _BM, 128), lambda i: (0, i, 0)),
                  pl.BlockSpec((_BM, 1), lambda i: (i, 0)),
                  pl.BlockSpec((1, H), lambda i: (0, 0))],
        out_specs=pl.BlockSpec((_BM, H), lambda i: (i, 0)),
        out_shape=jax.ShapeDtypeStruct((n, H), jnp.float32),
    )(acc, hb, dinv, bias)


def _row_chunks(total, step):
    out, off = [], 0
    while off < total:
        out.append((off, min(step, total - off)))
        off += step
    return out


def _sc_edge(hb, vals, src3, dst3, n, mode):
    NBn = hb.shape[0]
    NB = NBn // n
    NPB = NB // _NC
    NCH = src3.shape[1]
    NP = n + _JUNK
    RPT = NP // _NT
    gat = mode in ("gat", "gat_deg")
    with_deg = mode == "gat_deg"
    nv = len(vals)

    mesh = plsc.VectorSubcoreMesh(core_axis_name="c", subcore_axis_name="s")

    out_type = [jax.ShapeDtypeStruct((NB * NP, 128), jnp.float32)]
    if gat:
        out_type.append(jax.ShapeDtypeStruct((NP,), jnp.float32))
    if with_deg:
        out_type.append(jax.ShapeDtypeStruct((NP,), jnp.float32))

    scratch = ([pltpu.VMEM((NCH, _CK), jnp.float32)] +
               [pltpu.VMEM((1, _CK), jnp.int32) for _ in range(3)] +
               [pltpu.VMEM((1, _CK), jnp.float32) for _ in range(4)] +
               [pltpu.VMEM((_CK, 128), jnp.float32),
                pltpu.VMEM((128,), jnp.float32)] +
               [pltpu.VMEM_SHARED((NP,), jnp.float32) for _ in range(nv)] +
               [pltpu.VMEM_SHARED((NP, 128), jnp.float32),
                pltpu.VMEM_SHARED((NP,), jnp.float32)])

    def body(hb_ref, *refs):
        i = 0
        val_refs = refs[i:i + nv]; i += nv
        src_ref, dst_ref = refs[i], refs[i + 1]; i += 2
        acc_ref = refs[i]; i += 1
        den_ref = deg_ref = None
        if gat:
            den_ref = refs[i]; i += 1
        if with_deg:
            deg_ref = refs[i]; i += 1
        p_v = refs[i]; i += 1
        src_c, dst_c, srco_c = refs[i:i + 3]; i += 3
        g1, g2, g3, ones_c = refs[i:i + 4]; i += 4
        rowbuf, bounce_v = refs[i:i + 2]; i += 2
        val_sp = refs[i:i + nv]; i += nv
        acc_sp, den_sp = refs[i:i + 2]

        c = lax.axis_index("c")
        s = lax.axis_index("s")
        row0 = s * RPT
        zeros16 = jnp.zeros((16,), jnp.float32)
        ones16 = jnp.full((16,), 1.0, jnp.float32)
        rslices = _row_chunks(RPT, 128)

        for vr, vs in zip(val_refs, val_sp):
            for off, sz in rslices:
                pltpu.sync_copy(vr.at[pl.ds(row0 + off, sz)],
                                bounce_v.at[pl.ds(0, sz)])
                pltpu.sync_copy(bounce_v.at[pl.ds(0, sz)],
                                vs.at[pl.ds(row0 + off, sz)])

        def zrow(j, _):
            for u in range(8):
                rowbuf[j, pl.ds(16 * u, 16)] = zeros16
            return 0
        lax.fori_loop(0, 128, zrow, 0)
        for u in range(8):
            ones_c[0, pl.ds(16 * u, 16)] = ones16
        if gat or with_deg:
            for off, sz in rslices:
                pltpu.sync_copy(rowbuf.at[0, pl.ds(0, sz)],
                                den_sp.at[pl.ds(row0 + off, sz)])
        plsc.subcore_barrier()

        def pchunk(ch, _):
            pltpu.sync_copy(src_ref.at[s, ch], src_c.at[0])
            pltpu.sync_copy(dst_ref.at[s, ch], dst_c.at[0])
            if gat:
                pltpu.sync_copy(val_sp[0].at[src_c.at[0]], g1.at[0])
                pltpu.sync_copy(val_sp[1].at[dst_c.at[0]], g2.at[0])
                pltpu.sync_copy(val_sp[2].at[dst_c.at[0]], g3.at[0])
                for u in range(8):
                    sl = pl.ds(16 * u, 16)
                    t = g1[0, sl] + g2[0, sl]
                    p_v[ch, sl] = jnp.exp(_leaky(t) - g3[0, sl])
            else:
                pltpu.sync_copy(val_sp[0].at[src_c.at[0]], g1.at[0])
                pltpu.sync_copy(val_sp[0].at[dst_c.at[0]], g2.at[0])
                for u in range(8):
                    sl = pl.ds(16 * u, 16)
                    p_v[ch, sl] = g1[0, sl] * g2[0, sl]
            if gat:
                @pl.when(c == 0)
                def _():
                    pltpu.sync_copy(p_v.at[ch], den_sp.at[dst_c.at[0]],
                                    add=True)
            return 0
        lax.fori_loop(0, NCH, pchunk, 0)

        if gat:
            plsc.subcore_barrier()
            for off, sz in rslices:
                @pl.when(c == 0)
                def _(off=off, sz=sz):
                    pltpu.sync_copy(den_sp.at[pl.ds(row0 + off, sz)],
                                    bounce_v.at[pl.ds(0, sz)])
                    pltpu.sync_copy(bounce_v.at[pl.ds(0, sz)],
                                    den_ref.at[pl.ds(row0 + off, sz)])

        if with_deg:
            plsc.subcore_barrier()
            for off, sz in rslices:
                pltpu.sync_copy(rowbuf.at[0, pl.ds(0, sz)],
                                den_sp.at[pl.ds(row0 + off, sz)])
            plsc.subcore_barrier()

            @pl.when(c == 0)
            def _():
                def gchunk(ch, _):
                    pltpu.sync_copy(dst_ref.at[s, ch], dst_c.at[0])
                    pltpu.sync_copy(ones_c.at[0], den_sp.at[dst_c.at[0]],
                                    add=True)
                    return 0
                lax.fori_loop(0, NCH, gchunk, 0)
            plsc.subcore_barrier()
            for off, sz in rslices:
                @pl.when(c == 0)
                def _(off=off, sz=sz):
                    pltpu.sync_copy(den_sp.at[pl.ds(row0 + off, sz)],
                                    bounce_v.at[pl.ds(0, sz)])
                    pltpu.sync_copy(bounce_v.at[pl.ds(0, sz)],
                                    deg_ref.at[pl.ds(row0 + off, sz)])

        for bi in range(NPB):
            blk = c * NPB + bi
            boffn = blk * n

            def zrow2(j, _):
                for u in range(8):
                    rowbuf[j, pl.ds(16 * u, 16)] = zeros16
                return 0
            lax.fori_loop(0, 128, zrow2, 0)
            for off, sz in rslices:
                pltpu.sync_copy(rowbuf.at[pl.ds(0, sz)],
                                acc_sp.at[pl.ds(row0 + off, sz)])
            plsc.subcore_barrier()

            def mchunk(ch, _):
                pltpu.sync_copy(src_ref.at[s, ch], src_c.at[0])
                pltpu.sync_copy(dst_ref.at[s, ch], dst_c.at[0])
                for u in range(8):
                    sl = pl.ds(16 * u, 16)
                    srco_c[0, sl] = src_c[0, sl] + boffn
                pltpu.sync_copy(hb_ref.at[srco_c.at[0]], rowbuf)

                def srow(g, _):
                    pv16 = p_v[ch, pl.ds(16 * g, 16)]
                    for l in range(16):
                        j = 16 * g + l
                        pj = pv16[l]
                        for u in range(8):
                            rowbuf[j, pl.ds(16 * u, 16)] = (
                                rowbuf[j, pl.ds(16 * u, 16)] * pj)
                    return 0
                lax.fori_loop(0, _CK // 16, srow, 0)
                pltpu.sync_copy(rowbuf, acc_sp.at[dst_c.at[0]], add=True)
                return 0
            lax.fori_loop(0, NCH, mchunk, 0)
            plsc.subcore_barrier()

            boffp = blk * NP
            for off, sz in rslices:
                pltpu.sync_copy(acc_sp.at[pl.ds(row0 + off, sz)],
                                rowbuf.at[pl.ds(0, sz)])
                pltpu.sync_copy(rowbuf.at[pl.ds(0, sz)],
                                acc_ref.at[pl.ds(boffp + row0 + off, sz)])
            plsc.subcore_barrier()

    fn = pl.kernel(body, out_type=out_type, mesh=mesh,
                   scratch_types=scratch,
                   compiler_params=pltpu.CompilerParams(
                       needs_layout_passes=False))
    res = fn(hb, *vals, src3, dst3)
    if not isinstance(res, (tuple, list)):
        res = (res,)
    accf = res[0].reshape(NB, NP, 128)
    den = res[1] if gat else None
    deg = res[2] if with_deg else None
    return accf, den, deg


def _gat_layer(x, W, a_s, a_d, b, src3, dst3, n, with_deg):
    a2 = jnp.stack([a_s, a_d], axis=1)
    hb, sd = _proj(x, W, a2)
    cp = _softmax_prep(sd)
    asrc = sd[:, 0]
    adst = sd[:, 1]
    cdst = cp[:, 0]
    pself = cp[:, 1:2]
    NB = hb.shape[0]
    hflat = hb.reshape(NB * n, 128)
    pad = ((0, _JUNK),)
    acc, den, deg = _sc_edge(
        hflat,
        (jnp.pad(asrc, pad), jnp.pad(adst, pad), jnp.pad(cdst, pad)),
        src3, dst3, n, "gat_deg" if with_deg else "gat")
    accn = acc[:, :n, :]
    denn = den[:n].reshape(n, 1)
    bias = b.reshape(1, -1)
    xn = _combine_gat(accn, hb, denn, pself, bias)
    return xn, deg


def kernel(x, edge_index, W1, a_src1, a_dst1, b1, W2, a_src2, a_dst2, b2,
           W3, a_src3, a_dst3, b3, W4, b4):
    n = x.shape[0]
    e = edge_index.shape[1]
    src = edge_index[0].astype(jnp.int32)
    dst = edge_index[1].astype(jnp.int32)

    ept = -(-e // _NT)
    nch = -(-ept // _CK)
    tot = _NT * nch * _CK
    padlen = tot - e
    ar = jnp.arange(padlen, dtype=jnp.int32)
    src_p = jnp.concatenate([src, ar % n])
    dst_p = jnp.concatenate([dst, n + (ar % _JUNK)])
    src3 = src_p.reshape(_NT, nch, _CK)
    dst3 = dst_p.reshape(_NT, nch, _CK)

    h, deg = _gat_layer(x, W1, a_src1, a_dst1, b1, src3, dst3, n, True)
    h, _ = _gat_layer(h, W2, a_src2, a_dst2, b2, src3, dst3, n, False)
    h, _ = _gat_layer(h, W3, a_src3, a_dst3, b3, src3, dst3, n, False)

    dinv = _dinv_kernel(deg[:n].reshape(n, 1))
    h4b, _ = _proj(h, W4, None)
    NB4 = h4b.shape[0]
    acc4, _, _ = _sc_edge(h4b.reshape(NB4 * n, 128),
                          (jnp.pad(dinv[:, 0], ((0, _JUNK),)),),
                          src3, dst3, n, "gcn")
    z = _combine_gcn(acc4[:, :n, :], h4b, dinv, b4.reshape(1, -1))
    return z


# speedup vs baseline: 11.1168x; 1.4447x over previous
import functools

import jax
import jax.numpy as jnp
from jax import lax
from jax.experimental import pallas as pl
from jax.experimental.pallas import tpu as pltpu
from jax.experimental.pallas import tpu_sc as plsc

_NT = 16
_NC = 2
_CK = 128
_JUNK = 112
_BM = 400


def _leaky(x):
    return jnp.maximum(x, 0.2 * x)


def _proj(x, W, a2):
    n, K = x.shape
    H = W.shape[1]
    NB = H // 128
    with_sd = a2 is not None

    def body(x_ref, w_ref, *rest):
        if with_sd:
            a_ref, hb_ref, sd_ref = rest
        else:
            (hb_ref,) = rest
        h = jnp.dot(x_ref[...], w_ref[...], preferred_element_type=jnp.float32)
        for b in range(NB):
            hb_ref[b, :, :] = h[:, b * 128:(b + 1) * 128]
        if with_sd:
            sd_ref[...] = jnp.dot(h, a_ref[...],
                                  preferred_element_type=jnp.float32)

    in_specs = [pl.BlockSpec((_BM, K), lambda i: (i, 0)),
                pl.BlockSpec((K, H), lambda i: (0, 0))]
    out_specs = [pl.BlockSpec((NB, _BM, 128), lambda i: (0, i, 0))]
    out_shape = [jax.ShapeDtypeStruct((NB, n, 128), jnp.float32)]
    args = [x, W]
    if with_sd:
        in_specs.append(pl.BlockSpec((H, 2), lambda i: (0, 0)))
        out_specs.append(pl.BlockSpec((_BM, 2), lambda i: (i, 0)))
        out_shape.append(jax.ShapeDtypeStruct((n, 2), jnp.float32))
        args.append(a2)
    res = pl.pallas_call(
        body, grid=(n // _BM,), in_specs=in_specs, out_specs=out_specs,
        out_shape=out_shape)(*args)
    return (res[0], res[1]) if with_sd else (res[0], None)


def _softmax_prep(sd):
    n = sd.shape[0]

    def body(sd_ref, ps_ref, mx_ref):
        asrc = sd_ref[:, 0:1]
        adst = sd_ref[:, 1:2]
        m = jnp.max(asrc)
        cdst = _leaky(m + adst)
        ps_ref[...] = jnp.exp(_leaky(asrc + adst) - cdst)
        mx_ref[...] = jnp.full((1, 128), m, jnp.float32)

    return pl.pallas_call(
        body, grid=(1,),
        in_specs=[pl.BlockSpec((n, 2), lambda i: (0, 0))],
        out_specs=[pl.BlockSpec((n, 1), lambda i: (0, 0)),
                   pl.BlockSpec((1, 128), lambda i: (0, 0))],
        out_shape=[jax.ShapeDtypeStruct((n, 1), jnp.float32),
                   jax.ShapeDtypeStruct((1, 128), jnp.float32)])(sd)


def _combine_gat(acc, hb, den, pself, bias):
    NB, n, _ = acc.shape
    H = NB * 128

    def body(acc_ref, hb_ref, den_ref, ps_ref, b_ref, o_ref):
        a = jnp.concatenate([acc_ref[b] for b in range(NB)], axis=1)
        h = jnp.concatenate([hb_ref[b] for b in range(NB)], axis=1)
        ps = ps_ref[...]
        dtot = den_ref[...] + ps
        o_ref[...] = jax.nn.relu((a + ps * h) / dtot + b_ref[...])

    return pl.pallas_call(
        body, grid=(n // _BM,),
        in_specs=[pl.BlockSpec((NB, _BM, 128), lambda i: (0, i, 0)),
                  pl.BlockSpec((NB, _BM, 128), lambda i: (0, i, 0)),
                  pl.BlockSpec((_BM, 1), lambda i: (i, 0)),
                  pl.BlockSpec((_BM, 1), lambda i: (i, 0)),
                  pl.BlockSpec((1, H), lambda i: (0, 0))],
        out_specs=pl.BlockSpec((_BM, H), lambda i: (i, 0)),
        out_shape=jax.ShapeDtypeStruct((n, H), jnp.float32),
    )(acc, hb, den, pself, bias)


def _dinv_kernel(deg):
    n = deg.shape[0]

    def body(d_ref, o_ref):
        o_ref[...] = lax.rsqrt(d_ref[...] + 1.0)

    return pl.pallas_call(
        body, grid=(1,),
        in_specs=[pl.BlockSpec((n, 1), lambda i: (0, 0))],
        out_specs=pl.BlockSpec((n, 1), lambda i: (0, 0)),
        out_shape=jax.ShapeDtypeStruct((n, 1), jnp.float32))(deg)


def _combine_gcn(acc, hb, dinv, bias):
    NB, n, _ = acc.shape
    H = NB * 128

    def body(acc_ref, hb_ref, di_ref, b_ref, o_ref):
        a = jnp.concatenate([acc_ref[b] for b in range(NB)], axis=1)
        h = jnp.concatenate([hb_ref[b] for b in range(NB)], axis=1)
        di = di_ref[...]
        o_ref[...] = a + (di * di) * h + b_ref[...]

    return pl.pallas_call(
        body, grid=(n // _BM,),
        in_specs=[pl.BlockSpec((NB, _BM, 128), lambda i: (0, i, 0)),
                  pl.BlockSpec((NB, _BM, 128), lambda i: (0, i, 0)),
                  pl.BlockSpec((_BM, 1), lambda i: (i, 0)),
                  pl.BlockSpec((1, H), lambda i: (0, 0))],
        out_specs=pl.BlockSpec((_BM, H), lambda i: (i, 0)),
        out_shape=jax.ShapeDtypeStruct((n, H), jnp.float32),
    )(acc, hb, dinv, bias)


def _row_chunks(total, step):
    out, off = [], 0
    while off < total:
        out.append((off, min(step, total - off)))
        off += step
    return out


def _sc_edge(hb, vals, mx, src3, dst3, dstq4, n, mode, nch):
    NBn = hb.shape[0]
    NB = NBn // n
    NPB = NB // _NC
    NP = n + _JUNK
    RPT = NP // _NT
    gat = mode in ("gat", "gat_deg")
    with_deg = mode == "gat_deg"
    nv = len(vals)
    NPAIR = nch // 2

    mesh = plsc.VectorSubcoreMesh(core_axis_name="c", subcore_axis_name="s")

    out_type = [jax.ShapeDtypeStruct((NB * NP, 128), jnp.float32)]
    if gat:
        out_type.append(jax.ShapeDtypeStruct((NP,), jnp.float32))
    if with_deg:
        out_type.append(jax.ShapeDtypeStruct((NP,), jnp.float32))

    def vm(shape, dt):
        return pltpu.VMEM(shape, dt)

    scratch = (
        [vm((128,), jnp.int32) for _ in range(2)] +
        [vm((128,), jnp.int32) for _ in range(2)] +
        [vm((128,), jnp.int32) for _ in range(2)] +
        [vm((4, 32), jnp.int32) for _ in range(2)] +
        [vm((128,), jnp.float32) for _ in range(2)] +
        [vm((128,), jnp.float32) for _ in range(2)] +
        [vm((4, 32), jnp.float32) for _ in range(2)] +
        [vm((128, 128), jnp.float32) for _ in range(2)] +
        [vm((32,), jnp.float32),
         vm((128,), jnp.float32),
         vm((128,), jnp.float32)] +
        [pltpu.SemaphoreType.DMA for _ in range(6)] +
        [pltpu.VMEM_SHARED((NP,), jnp.float32) for _ in range(nv)] +
        [pltpu.VMEM_SHARED((NP, 128), jnp.float32),
         pltpu.VMEM_SHARED((NP,), jnp.float32)])

    def body(hb_ref, *refs):
        i = 0
        val_refs = refs[i:i + nv]; i += nv
        mx_ref = None
        if gat:
            mx_ref = refs[i]; i += 1
        src_ref, dst_ref, dstq_ref = refs[i:i + 3]; i += 3
        acc_ref = refs[i]; i += 1
        den_ref = deg_ref = None
        if gat:
            den_ref = refs[i]; i += 1
        if with_deg:
            deg_ref = refs[i]; i += 1
        src_c = refs[i:i + 2]; i += 2
        dst_c = refs[i:i + 2]; i += 2
        srco_c = refs[i:i + 2]; i += 2
        dst_q = refs[i:i + 2]; i += 2
        g1 = refs[i:i + 2]; i += 2
        g2 = refs[i:i + 2]; i += 2
        p_c = refs[i:i + 2]; i += 2
        rowbuf = refs[i:i + 2]; i += 2
        ones_c, mx_c, bounce_v = refs[i:i + 3]; i += 3
        sem_row = refs[i:i + 2]; i += 2
        sem_g = refs[i:i + 2]; i += 2
        sem_sc = refs[i:i + 2]; i += 2
        val_sp = refs[i:i + nv]; i += nv
        acc_sp, den_sp = refs[i:i + 2]

        c = lax.axis_index("c")
        s = lax.axis_index("s")
        row0 = s * RPT
        zeros16 = jnp.zeros((16,), jnp.float32)
        rslices = _row_chunks(RPT, 128)

        for vr, vs in zip(val_refs, val_sp):
            for off, sz in rslices:
                pltpu.sync_copy(vr.at[pl.ds(row0 + off, sz)],
                                bounce_v.at[pl.ds(0, sz)])
                pltpu.sync_copy(bounce_v.at[pl.ds(0, sz)],
                                vs.at[pl.ds(row0 + off, sz)])
        if gat:
            pltpu.sync_copy(mx_ref.at[0], mx_c)
        for u in range(2):
            ones_c[pl.ds(16 * u, 16)] = jnp.full((16,), 1.0, jnp.float32)

        def zero_rowbuf0(_j, _):
            for u in range(8):
                rowbuf[0][_j, pl.ds(16 * u, 16)] = zeros16
            return 0

        lax.fori_loop(0, 128, zero_rowbuf0, 0)
        if gat:
            for off, sz in rslices:
                pltpu.sync_copy(rowbuf[0].at[0, pl.ds(0, sz)],
                                den_sp.at[pl.ds(row0 + off, sz)])
        plsc.subcore_barrier()

        def idx_load(ch, sl):
            pltpu.sync_copy(src_ref.at[s, ch], src_c[sl])
            pltpu.sync_copy(dst_ref.at[s, ch], dst_c[sl])
            pltpu.sync_copy(dstq_ref.at[s, ch], dst_q[sl])

        def gather_descs(sl):
            vsl = val_sp[1] if gat else val_sp[0]
            return (pltpu.make_async_copy(hb_ref.at[srco_c[sl]], rowbuf[sl],
                                          sem_row[sl]),
                    pltpu.make_async_copy(val_sp[0].at[src_c[sl]], g1[sl],
                                          sem_g[sl]),
                    pltpu.make_async_copy(vsl.at[dst_c[sl]], g2[sl],
                                          sem_g[sl]))

        def issue_gathers(sl):
            for d in gather_descs(sl):
                d.start()

        def wait_gathers(sl):
            for d in gather_descs(sl):
                d.wait()

        def drain_sc(sl):
            for k in range(4):
                pltpu.make_async_copy(
                    rowbuf[sl].at[pl.ds(32 * k, 32)],
                    acc_sp.at[dst_q[sl].at[k]], sem_sc[sl]).wait()

        def compute_srco(sl, boffn):
            for u in range(8):
                srco_c[sl][pl.ds(16 * u, 16)] = (
                    src_c[sl][pl.ds(16 * u, 16)] + boffn)

        def process(sl, den_pass):
            wait_gathers(sl)
            for k in range(4):
                for h in range(2):
                    fsl = pl.ds(32 * k + 16 * h, 16)
                    if gat:
                        t = g1[sl][fsl] + g2[sl][fsl]
                        u_ = mx_c[pl.ds(0, 16)] + g2[sl][fsl]
                        p = jnp.exp(_leaky(t) - _leaky(u_))
                    else:
                        p = g1[sl][fsl] * g2[sl][fsl]
                    p_c[sl][k, pl.ds(16 * h, 16)] = p
            if den_pass:
                @pl.when(c == 0)
                def _():
                    for k in range(4):
                        pltpu.sync_copy(p_c[sl].at[k],
                                        den_sp.at[dst_q[sl].at[k]],
                                        add=True)
            def sgrp(g, _):
                pv16 = p_c[sl][g // 2, pl.ds(16 * (g % 2), 16)]
                for l in range(16):
                    j = 16 * g + l
                    pj = pv16[l]
                    for u in range(8):
                        rowbuf[sl][j, pl.ds(16 * u, 16)] = (
                            rowbuf[sl][j, pl.ds(16 * u, 16)] * pj)
                return 0
            lax.fori_loop(0, 8, sgrp, 0)
            for k in range(4):
                pltpu.make_async_copy(
                    rowbuf[sl].at[pl.ds(32 * k, 32)],
                    acc_sp.at[dst_q[sl].at[k]], sem_sc[sl]).start(add=True)

        for bi in range(NPB):
            blk = c * NPB + bi
            boffn = blk * n
            den_pass = gat and bi == 0

            if bi > 0:
                def zero_rb(_j, _):
                    for u in range(8):
                        rowbuf[0][_j, pl.ds(16 * u, 16)] = zeros16
                    return 0
                lax.fori_loop(0, 128, zero_rb, 0)
            for off, sz in rslices:
                pltpu.sync_copy(rowbuf[0].at[pl.ds(0, sz)],
                                acc_sp.at[pl.ds(row0 + off, sz)])
            plsc.subcore_barrier()

            idx_load(0, 0)
            compute_srco(0, boffn)
            issue_gathers(0)
            idx_load(nch, 1)
            for k in range(4):
                pltpu.make_async_copy(
                    rowbuf[1].at[pl.ds(32 * k, 32)],
                    acc_sp.at[dst_q[1].at[k]], sem_sc[1]).start(add=True)

            def pair_body(g, _):
                a = 2 * g
                drain_sc(1)
                idx_load(a + 1, 1)
                compute_srco(1, boffn)
                issue_gathers(1)
                process(0, den_pass)
                process(1, den_pass)
                drain_sc(0)
                idx_load(a + 2, 0)
                compute_srco(0, boffn)
                issue_gathers(0)
                return 0
            lax.fori_loop(0, NPAIR, pair_body, 0)

            drain_sc(1)
            wait_gathers(0)
            plsc.subcore_barrier()

            if den_pass:
                for off, sz in rslices:
                    @pl.when(c == 0)
                    def _(off=off, sz=sz):
                        pltpu.sync_copy(den_sp.at[pl.ds(row0 + off, sz)],
                                        bounce_v.at[pl.ds(0, sz)])
                        pltpu.sync_copy(bounce_v.at[pl.ds(0, sz)],
                                        den_ref.at[pl.ds(row0 + off, sz)])
            boffp = blk * NP
            for off, sz in rslices:
                pltpu.sync_copy(acc_sp.at[pl.ds(row0 + off, sz)],
                                rowbuf[1].at[pl.ds(0, sz)])
                pltpu.sync_copy(rowbuf[1].at[pl.ds(0, sz)],
                                acc_ref.at[pl.ds(boffp + row0 + off, sz)])
            plsc.subcore_barrier()

        if with_deg:
            def zero_rb3(_j, _):
                for u in range(8):
                    rowbuf[0][_j, pl.ds(16 * u, 16)] = zeros16
                return 0
            lax.fori_loop(0, 128, zero_rb3, 0)
            for off, sz in rslices:
                pltpu.sync_copy(rowbuf[0].at[0, pl.ds(0, sz)],
                                den_sp.at[pl.ds(row0 + off, sz)])
            plsc.subcore_barrier()

            @pl.when(c == 0)
            def _():
                def gchunk(ch, _):
                    pltpu.sync_copy(dstq_ref.at[s, ch], dst_q[0])
                    for k in range(4):
                        pltpu.sync_copy(ones_c,
                                        den_sp.at[dst_q[0].at[k]],
                                        add=True)
                    return 0
                lax.fori_loop(0, nch, gchunk, 0)
            plsc.subcore_barrier()
            for off, sz in rslices:
                @pl.when(c == 0)
                def _(off=off, sz=sz):
                    pltpu.sync_copy(den_sp.at[pl.ds(row0 + off, sz)],
                                    bounce_v.at[pl.ds(0, sz)])
                    pltpu.sync_copy(bounce_v.at[pl.ds(0, sz)],
                                    deg_ref.at[pl.ds(row0 + off, sz)])

    fn = pl.kernel(body, out_type=out_type, mesh=mesh,
                   scratch_types=scratch,
                   compiler_params=pltpu.CompilerParams(
                       needs_layout_passes=False))
    args = [hb] + list(vals)
    if gat:
        args.append(mx)
    args += [src3, dst3, dstq4]
    res = fn(*args)
    if not isinstance(res, (tuple, list)):
        res = (res,)
    accf = res[0].reshape(NB, NP, 128)
    den = res[1] if gat else None
    deg = res[2] if with_deg else None
    return accf, den, deg


def _gat_layer(x, W, a_s, a_d, b, src3, dst3, dstq4, n, nch, with_deg):
    a2 = jnp.stack([a_s, a_d], axis=1)
    hb, sd = _proj(x, W, a2)
    pself, mx = _softmax_prep(sd)
    asrc = sd[:, 0]
    adst = sd[:, 1]
    NB = hb.shape[0]
    hflat = hb.reshape(NB * n, 128)
    pad = ((0, _JUNK),)
    acc, den, deg = _sc_edge(
        hflat, (jnp.pad(asrc, pad), jnp.pad(adst, pad)), mx,
        src3, dst3, dstq4, n, "gat_deg" if with_deg else "gat", nch)
    accn = acc[:, :n, :]
    denn = den[:n].reshape(n, 1)
    bias = b.reshape(1, -1)
    xn = _combine_gat(accn, hb, denn, pself, bias)
    return xn, deg


def kernel(x, edge_index, W1, a_src1, a_dst1, b1, W2, a_src2, a_dst2, b2,
           W3, a_src3, a_dst3, b3, W4, b4):
    n = x.shape[0]
    e = edge_index.shape[1]
    src = edge_index[0].astype(jnp.int32)
    dst = edge_index[1].astype(jnp.int32)

    ept = -(-e // _NT)
    nch = -(-ept // _CK)
    nch += nch % 2
    ncha = nch + 2
    padlen = _NT * nch * _CK - e
    ar = jnp.arange(padlen, dtype=jnp.int32)
    src_p = jnp.concatenate([src, ar % n]).reshape(_NT, nch, _CK)
    dst_p = jnp.concatenate([dst, n + (ar % _JUNK)]).reshape(_NT, nch, _CK)
    arj = jnp.arange(_NT * 2 * _CK, dtype=jnp.int32)
    srcj = (arj % n).reshape(_NT, 2, _CK)
    dstj = (n + (arj % _JUNK)).reshape(_NT, 2, _CK)
    src3 = jnp.concatenate([src_p, srcj], axis=1)
    dst3 = jnp.concatenate([dst_p, dstj], axis=1)
    dstq4 = dst3.reshape(_NT, ncha, 4, 32)

    h, deg = _gat_layer(x, W1, a_src1, a_dst1, b1, src3, dst3, dstq4, n,
                        nch, True)
    h, _ = _gat_layer(h, W2, a_src2, a_dst2, b2, src3, dst3, dstq4, n,
                      nch, False)
    h, _ = _gat_layer(h, W3, a_src3, a_dst3, b3, src3, dst3, dstq4, n,
                      nch, False)

    dinv = _dinv_kernel(deg[:n].reshape(n, 1))
    h4b, _ = _proj(h, W4, None)
    NB4 = h4b.shape[0]
    acc4, _, _ = _sc_edge(h4b.reshape(NB4 * n, 128),
                          (jnp.pad(dinv[:, 0], ((0, _JUNK),)),), None,
                          src3, dst3, dstq4, n, "gcn", nch)
    z = _combine_gcn(acc4[:, :n, :], h4b, dinv, b4.reshape(1, -1))
    return z


# speedup vs baseline: 13.4790x; 1.2125x over previous
import functools

import jax
import jax.numpy as jnp
from jax import lax
from jax.experimental import pallas as pl
from jax.experimental.pallas import tpu as pltpu
from jax.experimental.pallas import tpu_sc as plsc

_NT = 16
_NC = 2
_CK = 128
_JUNK = 112
_BM = 400


def _leaky(x):
    return jnp.maximum(x, 0.2 * x)


def _proj(x, W, a2):
    n, K = x.shape
    H = W.shape[1]
    NB = H // 128
    with_sd = a2 is not None

    def body(x_ref, w_ref, *rest):
        if with_sd:
            a_ref, hb_ref, sd_ref = rest
        else:
            (hb_ref,) = rest
        h = jnp.dot(x_ref[...], w_ref[...], preferred_element_type=jnp.float32)
        for b in range(NB):
            hb_ref[b, :, :] = h[:, b * 128:(b + 1) * 128]
        if with_sd:
            sd_ref[...] = jnp.dot(h, a_ref[...],
                                  preferred_element_type=jnp.float32)

    in_specs = [pl.BlockSpec((_BM, K), lambda i: (i, 0)),
                pl.BlockSpec((K, H), lambda i: (0, 0))]
    out_specs = [pl.BlockSpec((NB, _BM, 128), lambda i: (0, i, 0))]
    out_shape = [jax.ShapeDtypeStruct((NB, n, 128), jnp.float32)]
    args = [x, W]
    if with_sd:
        in_specs.append(pl.BlockSpec((H, 2), lambda i: (0, 0)))
        out_specs.append(pl.BlockSpec((_BM, 2), lambda i: (i, 0)))
        out_shape.append(jax.ShapeDtypeStruct((n, 2), jnp.float32))
        args.append(a2)
    res = pl.pallas_call(
        body, grid=(n // _BM,), in_specs=in_specs, out_specs=out_specs,
        out_shape=out_shape)(*args)
    return (res[0], res[1]) if with_sd else (res[0], None)


def _softmax_prep(sd):
    n = sd.shape[0]

    def body(sd_ref, ps_ref, mx_ref):
        asrc = sd_ref[:, 0:1]
        adst = sd_ref[:, 1:2]
        m = jnp.max(asrc)
        cdst = _leaky(m + adst)
        ps_ref[...] = jnp.exp(_leaky(asrc + adst) - cdst)
        mx_ref[...] = jnp.full((1, 128), m, jnp.float32)

    return pl.pallas_call(
        body, grid=(1,),
        in_specs=[pl.BlockSpec((n, 2), lambda i: (0, 0))],
        out_specs=[pl.BlockSpec((n, 1), lambda i: (0, 0)),
                   pl.BlockSpec((1, 128), lambda i: (0, 0))],
        out_shape=[jax.ShapeDtypeStruct((n, 1), jnp.float32),
                   jax.ShapeDtypeStruct((1, 128), jnp.float32)])(sd)


def _combine_gat(acc, hb, den, pself, bias):
    NB, n, _ = acc.shape
    H = NB * 128

    def body(acc_ref, hb_ref, den_ref, ps_ref, b_ref, o_ref):
        a = jnp.concatenate([acc_ref[b] for b in range(NB)], axis=1)
        h = jnp.concatenate([hb_ref[b] for b in range(NB)], axis=1)
        ps = ps_ref[...]
        dtot = den_ref[...] + ps
        o_ref[...] = jax.nn.relu((a + ps * h) / dtot + b_ref[...])

    return pl.pallas_call(
        body, grid=(n // _BM,),
        in_specs=[pl.BlockSpec((NB, _BM, 128), lambda i: (0, i, 0)),
                  pl.BlockSpec((NB, _BM, 128), lambda i: (0, i, 0)),
                  pl.BlockSpec((_BM, 1), lambda i: (i, 0)),
                  pl.BlockSpec((_BM, 1), lambda i: (i, 0)),
                  pl.BlockSpec((1, H), lambda i: (0, 0))],
        out_specs=pl.BlockSpec((_BM, H), lambda i: (i, 0)),
        out_shape=jax.ShapeDtypeStruct((n, H), jnp.float32),
    )(acc, hb, den, pself, bias)


def _dinv_kernel(deg):
    n = deg.shape[0]

    def body(d_ref, o_ref):
        o_ref[...] = lax.rsqrt(d_ref[...] + 1.0)

    return pl.pallas_call(
        body, grid=(1,),
        in_specs=[pl.BlockSpec((n, 1), lambda i: (0, 0))],
        out_specs=pl.BlockSpec((n, 1), lambda i: (0, 0)),
        out_shape=jax.ShapeDtypeStruct((n, 1), jnp.float32))(deg)


def _combine_gcn(acc, hb, dinv, bias):
    NB, n, _ = acc.shape
    H = NB * 128

    def body(acc_ref, hb_ref, di_ref, b_ref, o_ref):
        a = jnp.concatenate([acc_ref[b] for b in range(NB)], axis=1)
        h = jnp.concatenate([hb_ref[b] for b in range(NB)], axis=1)
        di = di_ref[...]
        o_ref[...] = a + (di * di) * h + b_ref[...]

    return pl.pallas_call(
        body, grid=(n // _BM,),
        in_specs=[pl.BlockSpec((NB, _BM, 128), lambda i: (0, i, 0)),
                  pl.BlockSpec((NB, _BM, 128), lambda i: (0, i, 0)),
                  pl.BlockSpec((_BM, 1), lambda i: (i, 0)),
                  pl.BlockSpec((1, H), lambda i: (0, 0))],
        out_specs=pl.BlockSpec((_BM, H), lambda i: (i, 0)),
        out_shape=jax.ShapeDtypeStruct((n, H), jnp.float32),
    )(acc, hb, dinv, bias)


def _row_chunks(total, step):
    out, off = [], 0
    while off < total:
        out.append((off, min(step, total - off)))
        off += step
    return out


def _sc_edge(hb, vals, mx, src3, dst3, n, mode, nch):
    NBn = hb.shape[0]
    NB = NBn // n
    NPB = NB // _NC
    NP = n + _JUNK
    RPT = NP // _NT
    gat = mode in ("gat", "gat_deg")
    with_deg = mode == "gat_deg"
    nv = len(vals)
    NPAIR = nch // 2

    mesh = plsc.VectorSubcoreMesh(core_axis_name="c", subcore_axis_name="s")

    out_type = [jax.ShapeDtypeStruct((NB * NP, 128), jnp.float32)]
    if gat:
        out_type.append(jax.ShapeDtypeStruct((NP,), jnp.float32))
    if with_deg:
        out_type.append(jax.ShapeDtypeStruct((NP,), jnp.float32))

    def vm(shape, dt):
        return pltpu.VMEM(shape, dt)

    scratch = (
        [vm((128,), jnp.int32) for _ in range(2)] +
        [vm((1, 128), jnp.int32) for _ in range(2)] +
        [vm((128,), jnp.int32) for _ in range(2)] +
        [vm((128,), jnp.float32) for _ in range(2)] +
        [vm((128,), jnp.float32) for _ in range(2)] +
        [vm((1, 128), jnp.float32) for _ in range(2)] +
        [vm((128, 128), jnp.float32) for _ in range(2)] +
        [vm((1, 128), jnp.float32),
         vm((128,), jnp.float32),
         vm((128,), jnp.float32)] +
        [pltpu.SemaphoreType.DMA for _ in range(6)] +
        [pltpu.VMEM_SHARED((NP,), jnp.float32) for _ in range(nv)] +
        [pltpu.VMEM_SHARED((NP, 128), jnp.float32),
         pltpu.VMEM_SHARED((NP,), jnp.float32)])

    def body(hb_ref, *refs):
        i = 0
        val_refs = refs[i:i + nv]; i += nv
        mx_ref = None
        if gat:
            mx_ref = refs[i]; i += 1
        src_ref, dst_ref = refs[i:i + 2]; i += 2
        acc_ref = refs[i]; i += 1
        den_ref = deg_ref = None
        if gat:
            den_ref = refs[i]; i += 1
        if with_deg:
            deg_ref = refs[i]; i += 1
        src_c = refs[i:i + 2]; i += 2
        dst_c = refs[i:i + 2]; i += 2
        srco_c = refs[i:i + 2]; i += 2
        g1 = refs[i:i + 2]; i += 2
        g2 = refs[i:i + 2]; i += 2
        p_c = refs[i:i + 2]; i += 2
        rowbuf = refs[i:i + 2]; i += 2
        ones_c, mx_c, bounce_v = refs[i:i + 3]; i += 3
        sem_row = refs[i:i + 2]; i += 2
        sem_g = refs[i:i + 2]; i += 2
        sem_sc = refs[i:i + 2]; i += 2
        val_sp = refs[i:i + nv]; i += nv
        acc_sp, den_sp = refs[i:i + 2]

        c = lax.axis_index("c")
        s = lax.axis_index("s")
        row0 = s * RPT
        zeros16 = jnp.zeros((16,), jnp.float32)
        rslices = _row_chunks(RPT, 128)

        for vr, vs in zip(val_refs, val_sp):
            for off, sz in rslices:
                pltpu.sync_copy(vr.at[pl.ds(row0 + off, sz)],
                                bounce_v.at[pl.ds(0, sz)])
                pltpu.sync_copy(bounce_v.at[pl.ds(0, sz)],
                                vs.at[pl.ds(row0 + off, sz)])
        if gat:
            pltpu.sync_copy(mx_ref.at[0], mx_c)
        for u in range(8):
            ones_c[0, pl.ds(16 * u, 16)] = jnp.full((16,), 1.0, jnp.float32)

        def zero_rowbuf0(_j, _):
            for u in range(8):
                rowbuf[0][_j, pl.ds(16 * u, 16)] = zeros16
            return 0

        lax.fori_loop(0, 128, zero_rowbuf0, 0)
        if gat:
            for off, sz in rslices:
                pltpu.sync_copy(rowbuf[0].at[0, pl.ds(0, sz)],
                                den_sp.at[pl.ds(row0 + off, sz)])
        plsc.subcore_barrier()

        def idx_load(ch, sl):
            pltpu.sync_copy(src_ref.at[s, ch], src_c[sl])
            pltpu.sync_copy(dst_ref.at[s, ch], dst_c[sl].at[0])

        def gather_descs(sl):
            vsl = val_sp[1] if gat else val_sp[0]
            return (pltpu.make_async_copy(hb_ref.at[srco_c[sl]], rowbuf[sl],
                                          sem_row[sl]),
                    pltpu.make_async_copy(val_sp[0].at[src_c[sl]], g1[sl],
                                          sem_g[sl]),
                    pltpu.make_async_copy(vsl.at[dst_c[sl].at[0]], g2[sl],
                                          sem_g[sl]))

        def issue_gathers(sl):
            for d in gather_descs(sl):
                d.start()

        def wait_gathers(sl):
            for d in gather_descs(sl):
                d.wait()

        def drain_sc(sl):
            pltpu.make_async_copy(rowbuf[sl],
                                  acc_sp.at[dst_c[sl].at[0]],
                                  sem_sc[sl]).wait()

        def compute_srco(sl, boffn):
            for u in range(8):
                srco_c[sl][pl.ds(16 * u, 16)] = (
                    src_c[sl][pl.ds(16 * u, 16)] + boffn)

        def process(sl, den_pass):
            wait_gathers(sl)
            for g8 in range(8):
                fsl = pl.ds(16 * g8, 16)
                if gat:
                    t = g1[sl][fsl] + g2[sl][fsl]
                    u_ = mx_c[pl.ds(0, 16)] + g2[sl][fsl]
                    p = jnp.exp(_leaky(t) - _leaky(u_))
                else:
                    p = g1[sl][fsl] * g2[sl][fsl]
                p_c[sl][0, fsl] = p
            if den_pass:
                @pl.when(c == 0)
                def _():
                    pltpu.sync_copy(p_c[sl].at[0],
                                    den_sp.at[dst_c[sl].at[0]], add=True)
                if with_deg:
                    @pl.when(c == 1)
                    def _():
                        pltpu.sync_copy(ones_c.at[0],
                                        den_sp.at[dst_c[sl].at[0]],
                                        add=True)
            def sgrp(g, _):
                pv16 = p_c[sl][0, pl.ds(16 * g, 16)]
                for l in range(16):
                    j = 16 * g + l
                    pj = pv16[l]
                    for u in range(8):
                        rowbuf[sl][j, pl.ds(16 * u, 16)] = (
                            rowbuf[sl][j, pl.ds(16 * u, 16)] * pj)
                return 0
            lax.fori_loop(0, 8, sgrp, 0, unroll=2)
            pltpu.make_async_copy(rowbuf[sl],
                                  acc_sp.at[dst_c[sl].at[0]],
                                  sem_sc[sl]).start(add=True)

        for bi in range(NPB):
            blk = c * NPB + bi
            boffn = blk * n
            den_pass = gat and bi == 0

            if bi > 0:
                def zero_rb(_j, _):
                    for u in range(8):
                        rowbuf[0][_j, pl.ds(16 * u, 16)] = zeros16
                    return 0
                lax.fori_loop(0, 128, zero_rb, 0)
            for off, sz in rslices:
                pltpu.sync_copy(rowbuf[0].at[pl.ds(0, sz)],
                                acc_sp.at[pl.ds(row0 + off, sz)])
            plsc.subcore_barrier()

            idx_load(0, 0)
            compute_srco(0, boffn)
            issue_gathers(0)
            idx_load(nch, 1)
            pltpu.make_async_copy(rowbuf[1],
                                  acc_sp.at[dst_c[1].at[0]],
                                  sem_sc[1]).start(add=True)

            def pair_body(g, _):
                a = 2 * g
                drain_sc(1)
                idx_load(a + 1, 1)
                compute_srco(1, boffn)
                issue_gathers(1)
                process(0, den_pass)
                process(1, den_pass)
                drain_sc(0)
                idx_load(a + 2, 0)
                compute_srco(0, boffn)
                issue_gathers(0)
                return 0
            lax.fori_loop(0, NPAIR, pair_body, 0)

            drain_sc(1)
            wait_gathers(0)
            plsc.subcore_barrier()

            if den_pass:
                for off, sz in rslices:
                    @pl.when(c == 0)
                    def _(off=off, sz=sz):
                        pltpu.sync_copy(den_sp.at[pl.ds(row0 + off, sz)],
                                        bounce_v.at[pl.ds(0, sz)])
                        pltpu.sync_copy(bounce_v.at[pl.ds(0, sz)],
                                        den_ref.at[pl.ds(row0 + off, sz)])
                if with_deg:
                    for off, sz in rslices:
                        @pl.when(c == 1)
                        def _(off=off, sz=sz):
                            pltpu.sync_copy(den_sp.at[pl.ds(row0 + off, sz)],
                                            bounce_v.at[pl.ds(0, sz)])
                            pltpu.sync_copy(bounce_v.at[pl.ds(0, sz)],
                                            deg_ref.at[pl.ds(row0 + off, sz)])
            boffp = blk * NP
            for off, sz in rslices:
                pltpu.sync_copy(acc_sp.at[pl.ds(row0 + off, sz)],
                                rowbuf[1].at[pl.ds(0, sz)])
                pltpu.sync_copy(rowbuf[1].at[pl.ds(0, sz)],
                                acc_ref.at[pl.ds(boffp + row0 + off, sz)])
            plsc.subcore_barrier()

    fn = pl.kernel(body, out_type=out_type, mesh=mesh,
                   scratch_types=scratch,
                   compiler_params=pltpu.CompilerParams(
                       needs_layout_passes=False))
    args = [hb] + list(vals)
    if gat:
        args.append(mx)
    args += [src3, dst3]
    res = fn(*args)
    if not isinstance(res, (tuple, list)):
        res = (res,)
    accf = res[0].reshape(NB, NP, 128)
    den = res[1] if gat else None
    deg = res[2] if with_deg else None
    return accf, den, deg


def _gat_layer(x, W, a_s, a_d, b, src3, dst3, n, nch, with_deg):
    a2 = jnp.stack([a_s, a_d], axis=1)
    hb, sd = _proj(x, W, a2)
    pself, mx = _softmax_prep(sd)
    asrc = sd[:, 0]
    adst = sd[:, 1]
    NB = hb.shape[0]
    hflat = hb.reshape(NB * n, 128)
    pad = ((0, _JUNK),)
    acc, den, deg = _sc_edge(
        hflat, (jnp.pad(asrc, pad), jnp.pad(adst, pad)), mx,
        src3, dst3, n, "gat_deg" if with_deg else "gat", nch)
    accn = acc[:, :n, :]
    denn = den[:n].reshape(n, 1)
    bias = b.reshape(1, -1)
    xn = _combine_gat(accn, hb, denn, pself, bias)
    return xn, deg


def kernel(x, edge_index, W1, a_src1, a_dst1, b1, W2, a_src2, a_dst2, b2,
           W3, a_src3, a_dst3, b3, W4, b4):
    n = x.shape[0]
    e = edge_index.shape[1]
    src = edge_index[0].astype(jnp.int32)
    dst = edge_index[1].astype(jnp.int32)

    ept = -(-e // _NT)
    nch = -(-ept // _CK)
    nch += nch % 2
    ncha = nch + 2
    padlen = _NT * nch * _CK - e
    ar = jnp.arange(padlen, dtype=jnp.int32)
    src_p = jnp.concatenate([src, ar % n]).reshape(_NT, nch, _CK)
    dst_p = jnp.concatenate([dst, n + (ar % _JUNK)]).reshape(_NT, nch, _CK)
    arj = jnp.arange(_NT * 2 * _CK, dtype=jnp.int32)
    srcj = (arj % n).reshape(_NT, 2, _CK)
    dstj = (n + (arj % _JUNK)).reshape(_NT, 2, _CK)
    src3 = jnp.concatenate([src_p, srcj], axis=1)
    dst3 = jnp.concatenate([dst_p, dstj], axis=1)

    h, deg = _gat_layer(x, W1, a_src1, a_dst1, b1, src3, dst3, n,
                        nch, True)
    h, _ = _gat_layer(h, W2, a_src2, a_dst2, b2, src3, dst3, n,
                      nch, False)
    h, _ = _gat_layer(h, W3, a_src3, a_dst3, b3, src3, dst3, n,
                      nch, False)

    dinv = _dinv_kernel(deg[:n].reshape(n, 1))
    h4b, _ = _proj(h, W4, None)
    NB4 = h4b.shape[0]
    acc4, _, _ = _sc_edge(h4b.reshape(NB4 * n, 128),
                          (jnp.pad(dinv[:, 0], ((0, _JUNK),)),), None,
                          src3, dst3, n, "gcn", nch)
    z = _combine_gcn(acc4[:, :n, :], h4b, dinv, b4.reshape(1, -1))
    return z


# speedup vs baseline: 14.0162x; 1.0399x over previous
import functools

import jax
import jax.numpy as jnp
from jax import lax
from jax.experimental import pallas as pl
from jax.experimental.pallas import tpu as pltpu
from jax.experimental.pallas import tpu_sc as plsc

_NT = 16
_NC = 2
_CK = 128
_JUNK = 112
_BM = 400


def _leaky(x):
    return jnp.maximum(x, 0.2 * x)


def _proj(x, W, a2):
    n, K = x.shape
    H = W.shape[1]
    NB = H // 128
    with_sd = a2 is not None

    def body(x_ref, w_ref, *rest):
        if with_sd:
            a_ref, hb_ref, sd_ref = rest
        else:
            (hb_ref,) = rest
        h = jnp.dot(x_ref[...], w_ref[...], preferred_element_type=jnp.float32)
        for b in range(NB):
            hb_ref[b, :, :] = h[:, b * 128:(b + 1) * 128]
        if with_sd:
            sd_ref[...] = jnp.dot(h, a_ref[...],
                                  preferred_element_type=jnp.float32)

    in_specs = [pl.BlockSpec((_BM, K), lambda i: (i, 0)),
                pl.BlockSpec((K, H), lambda i: (0, 0))]
    out_specs = [pl.BlockSpec((NB, _BM, 128), lambda i: (0, i, 0))]
    out_shape = [jax.ShapeDtypeStruct((NB, n, 128), jnp.float32)]
    args = [x, W]
    if with_sd:
        in_specs.append(pl.BlockSpec((H, 2), lambda i: (0, 0)))
        out_specs.append(pl.BlockSpec((_BM, 2), lambda i: (i, 0)))
        out_shape.append(jax.ShapeDtypeStruct((n, 2), jnp.float32))
        args.append(a2)
    res = pl.pallas_call(
        body, grid=(n // _BM,), in_specs=in_specs, out_specs=out_specs,
        out_shape=out_shape)(*args)
    return (res[0], res[1]) if with_sd else (res[0], None)


def _softmax_prep(sd):
    n = sd.shape[0]

    def body(sd_ref, ps_ref, mx_ref):
        asrc = sd_ref[:, 0:1]
        adst = sd_ref[:, 1:2]
        m = jnp.max(asrc)
        cdst = _leaky(m + adst)
        ps_ref[...] = jnp.exp(_leaky(asrc + adst) - cdst)
        mx_ref[...] = jnp.full((1, 128), m, jnp.float32)

    return pl.pallas_call(
        body, grid=(1,),
        in_specs=[pl.BlockSpec((n, 2), lambda i: (0, 0))],
        out_specs=[pl.BlockSpec((n, 1), lambda i: (0, 0)),
                   pl.BlockSpec((1, 128), lambda i: (0, 0))],
        out_shape=[jax.ShapeDtypeStruct((n, 1), jnp.float32),
                   jax.ShapeDtypeStruct((1, 128), jnp.float32)])(sd)


def _combine_gat(acc, hb, den, pself, bias):
    NB, n, _ = acc.shape
    H = NB * 128

    def body(acc_ref, hb_ref, den_ref, ps_ref, b_ref, o_ref):
        a = jnp.concatenate([acc_ref[b] for b in range(NB)], axis=1)
        h = jnp.concatenate([hb_ref[b] for b in range(NB)], axis=1)
        ps = ps_ref[...]
        dtot = den_ref[...] + ps
        o_ref[...] = jax.nn.relu((a + ps * h) / dtot + b_ref[...])

    return pl.pallas_call(
        body, grid=(n // _BM,),
        in_specs=[pl.BlockSpec((NB, _BM, 128), lambda i: (0, i, 0)),
                  pl.BlockSpec((NB, _BM, 128), lambda i: (0, i, 0)),
                  pl.BlockSpec((_BM, 1), lambda i: (i, 0)),
                  pl.BlockSpec((_BM, 1), lambda i: (i, 0)),
                  pl.BlockSpec((1, H), lambda i: (0, 0))],
        out_specs=pl.BlockSpec((_BM, H), lambda i: (i, 0)),
        out_shape=jax.ShapeDtypeStruct((n, H), jnp.float32),
    )(acc, hb, den, pself, bias)


def _dinv_kernel(deg):
    n = deg.shape[0]

    def body(d_ref, o_ref):
        o_ref[...] = lax.rsqrt(d_ref[...] + 1.0)

    return pl.pallas_call(
        body, grid=(1,),
        in_specs=[pl.BlockSpec((n, 1), lambda i: (0, 0))],
        out_specs=pl.BlockSpec((n, 1), lambda i: (0, 0)),
        out_shape=jax.ShapeDtypeStruct((n, 1), jnp.float32))(deg)


def _combine_gcn(acc, hb, dinv, bias):
    NB, n, _ = acc.shape
    H = NB * 128

    def body(acc_ref, hb_ref, di_ref, b_ref, o_ref):
        a = jnp.concatenate([acc_ref[b] for b in range(NB)], axis=1)
        h = jnp.concatenate([hb_ref[b] for b in range(NB)], axis=1)
        di = di_ref[...]
        o_ref[...] = a + (di * di) * h + b_ref[...]

    return pl.pallas_call(
        body, grid=(n // _BM,),
        in_specs=[pl.BlockSpec((NB, _BM, 128), lambda i: (0, i, 0)),
                  pl.BlockSpec((NB, _BM, 128), lambda i: (0, i, 0)),
                  pl.BlockSpec((_BM, 1), lambda i: (i, 0)),
                  pl.BlockSpec((1, H), lambda i: (0, 0))],
        out_specs=pl.BlockSpec((_BM, H), lambda i: (i, 0)),
        out_shape=jax.ShapeDtypeStruct((n, H), jnp.float32),
    )(acc, hb, dinv, bias)


def _combine_proj(acc, hb, den, pself, bias, W, a2, deg):
    NB, n, _ = acc.shape
    H = NB * 128
    Hn = W.shape[1]
    NBn = Hn // 128
    with_sd = a2 is not None
    with_dinv = deg is not None

    def body(acc_ref, hb_ref, den_ref, ps_ref, b_ref, w_ref, *rest):
        i = 0
        a_ref = d_ref = None
        if with_sd:
            a_ref = rest[i]; i += 1
        if with_dinv:
            d_ref = rest[i]; i += 1
        hbn_ref = rest[i]; i += 1
        sd_ref = di_ref = None
        if with_sd:
            sd_ref = rest[i]; i += 1
        if with_dinv:
            di_ref = rest[i]; i += 1
        a = jnp.concatenate([acc_ref[b] for b in range(NB)], axis=1)
        h = jnp.concatenate([hb_ref[b] for b in range(NB)], axis=1)
        ps = ps_ref[...]
        dtot = den_ref[...] + ps
        X = jax.nn.relu((a + ps * h) / dtot + b_ref[...])
        hn = jnp.dot(X, w_ref[...], preferred_element_type=jnp.float32)
        for b in range(NBn):
            hbn_ref[b, :, :] = hn[:, b * 128:(b + 1) * 128]
        if with_sd:
            sd_ref[...] = jnp.dot(hn, a_ref[...],
                                  preferred_element_type=jnp.float32)
        if with_dinv:
            di_ref[...] = lax.rsqrt(d_ref[...] + 1.0)

    in_specs = [pl.BlockSpec((NB, _BM, 128), lambda i: (0, i, 0)),
                pl.BlockSpec((NB, _BM, 128), lambda i: (0, i, 0)),
                pl.BlockSpec((_BM, 1), lambda i: (i, 0)),
                pl.BlockSpec((_BM, 1), lambda i: (i, 0)),
                pl.BlockSpec((1, H), lambda i: (0, 0)),
                pl.BlockSpec((H, Hn), lambda i: (0, 0))]
    args = [acc, hb, den, pself, bias, W]
    out_specs = [pl.BlockSpec((NBn, _BM, 128), lambda i: (0, i, 0))]
    out_shape = [jax.ShapeDtypeStruct((NBn, n, 128), jnp.float32)]
    if with_sd:
        in_specs.append(pl.BlockSpec((Hn, 2), lambda i: (0, 0)))
        args.append(a2)
        out_specs.append(pl.BlockSpec((_BM, 2), lambda i: (i, 0)))
        out_shape.append(jax.ShapeDtypeStruct((n, 2), jnp.float32))
    if with_dinv:
        in_specs.append(pl.BlockSpec((_BM, 1), lambda i: (i, 0)))
        args.append(deg)
        out_specs.append(pl.BlockSpec((_BM, 1), lambda i: (i, 0)))
        out_shape.append(jax.ShapeDtypeStruct((n, 1), jnp.float32))
    res = pl.pallas_call(
        body, grid=(n // _BM,), in_specs=in_specs, out_specs=out_specs,
        out_shape=out_shape)(*args)
    hbn = res[0]
    sd = res[1] if with_sd else None
    dinv = res[-1] if with_dinv else None
    return hbn, sd, dinv


def _row_chunks(total, step):
    out, off = [], 0
    while off < total:
        out.append((off, min(step, total - off)))
        off += step
    return out


def _sc_edge(hb, vals, mx, src3, dst3, n, mode, nch):
    NBn = hb.shape[0]
    NB = NBn // n
    NPB = NB // _NC
    NP = n + _JUNK
    RPT = NP // _NT
    gat = mode in ("gat", "gat_deg")
    with_deg = mode == "gat_deg"
    nv = len(vals)
    NPAIR = nch // 2

    mesh = plsc.VectorSubcoreMesh(core_axis_name="c", subcore_axis_name="s")

    out_type = [jax.ShapeDtypeStruct((NB * NP, 128), jnp.float32)]
    if gat:
        out_type.append(jax.ShapeDtypeStruct((NP,), jnp.float32))
    if with_deg:
        out_type.append(jax.ShapeDtypeStruct((NP,), jnp.float32))

    def vm(shape, dt):
        return pltpu.VMEM(shape, dt)

    scratch = (
        [vm((128,), jnp.int32) for _ in range(2)] +
        [vm((1, 128), jnp.int32) for _ in range(2)] +
        [vm((128,), jnp.int32) for _ in range(2)] +
        [vm((128,), jnp.float32) for _ in range(2)] +
        [vm((128,), jnp.float32) for _ in range(2)] +
        [vm((1, 128), jnp.float32) for _ in range(2)] +
        [vm((128, 128), jnp.float32) for _ in range(2)] +
        [vm((1, 128), jnp.float32),
         vm((128,), jnp.float32),
         vm((128,), jnp.float32)] +
        [pltpu.SemaphoreType.DMA for _ in range(6)] +
        [pltpu.VMEM_SHARED((NP,), jnp.float32) for _ in range(nv)] +
        [pltpu.VMEM_SHARED((NP, 128), jnp.float32),
         pltpu.VMEM_SHARED((NP,), jnp.float32)])

    def body(hb_ref, *refs):
        i = 0
        val_refs = refs[i:i + nv]; i += nv
        mx_ref = None
        if gat:
            mx_ref = refs[i]; i += 1
        src_ref, dst_ref = refs[i:i + 2]; i += 2
        acc_ref = refs[i]; i += 1
        den_ref = deg_ref = None
        if gat:
            den_ref = refs[i]; i += 1
        if with_deg:
            deg_ref = refs[i]; i += 1
        src_c = refs[i:i + 2]; i += 2
        dst_c = refs[i:i + 2]; i += 2
        srco_c = refs[i:i + 2]; i += 2
        g1 = refs[i:i + 2]; i += 2
        g2 = refs[i:i + 2]; i += 2
        p_c = refs[i:i + 2]; i += 2
        rowbuf = refs[i:i + 2]; i += 2
        ones_c, mx_c, bounce_v = refs[i:i + 3]; i += 3
        sem_row = refs[i:i + 2]; i += 2
        sem_g = refs[i:i + 2]; i += 2
        sem_sc = refs[i:i + 2]; i += 2
        val_sp = refs[i:i + nv]; i += nv
        acc_sp, den_sp = refs[i:i + 2]

        c = lax.axis_index("c")
        s = lax.axis_index("s")
        row0 = s * RPT
        zeros16 = jnp.zeros((16,), jnp.float32)
        rslices = _row_chunks(RPT, 128)

        for vr, vs in zip(val_refs, val_sp):
            for off, sz in rslices:
                pltpu.sync_copy(vr.at[pl.ds(row0 + off, sz)],
                                bounce_v.at[pl.ds(0, sz)])
                pltpu.sync_copy(bounce_v.at[pl.ds(0, sz)],
                                vs.at[pl.ds(row0 + off, sz)])
        if gat:
            pltpu.sync_copy(mx_ref.at[0], mx_c)
        for u in range(8):
            ones_c[0, pl.ds(16 * u, 16)] = jnp.full((16,), 1.0, jnp.float32)

        def zero_rowbuf0(_j, _):
            for u in range(8):
                rowbuf[0][_j, pl.ds(16 * u, 16)] = zeros16
            return 0

        lax.fori_loop(0, 128, zero_rowbuf0, 0)
        if gat:
            for off, sz in rslices:
                pltpu.sync_copy(rowbuf[0].at[0, pl.ds(0, sz)],
                                den_sp.at[pl.ds(row0 + off, sz)])
        plsc.subcore_barrier()

        def idx_load(ch, sl):
            pltpu.sync_copy(src_ref.at[s, ch], src_c[sl])
            pltpu.sync_copy(dst_ref.at[s, ch], dst_c[sl].at[0])

        def gather_descs(sl):
            vsl = val_sp[1] if gat else val_sp[0]
            return (pltpu.make_async_copy(hb_ref.at[srco_c[sl]], rowbuf[sl],
                                          sem_row[sl]),
                    pltpu.make_async_copy(val_sp[0].at[src_c[sl]], g1[sl],
                                          sem_g[sl]),
                    pltpu.make_async_copy(vsl.at[dst_c[sl].at[0]], g2[sl],
                                          sem_g[sl]))

        def issue_gathers(sl):
            for d in gather_descs(sl):
                d.start()

        def wait_gathers(sl):
            for d in gather_descs(sl):
                d.wait()

        def drain_sc(sl):
            pltpu.make_async_copy(rowbuf[sl],
                                  acc_sp.at[dst_c[sl].at[0]],
                                  sem_sc[sl]).wait()

        def compute_srco(sl, boffn):
            for u in range(8):
                srco_c[sl][pl.ds(16 * u, 16)] = (
                    src_c[sl][pl.ds(16 * u, 16)] + boffn)

        def process(sl, den_pass):
            wait_gathers(sl)
            for g8 in range(8):
                fsl = pl.ds(16 * g8, 16)
                if gat:
                    t = g1[sl][fsl] + g2[sl][fsl]
                    u_ = mx_c[pl.ds(0, 16)] + g2[sl][fsl]
                    p = jnp.exp(_leaky(t) - _leaky(u_))
                else:
                    p = g1[sl][fsl] * g2[sl][fsl]
                p_c[sl][0, fsl] = p
            if den_pass:
                @pl.when(c == 0)
                def _():
                    pltpu.sync_copy(p_c[sl].at[0],
                                    den_sp.at[dst_c[sl].at[0]], add=True)
                if with_deg:
                    @pl.when(c == 1)
                    def _():
                        pltpu.sync_copy(ones_c.at[0],
                                        den_sp.at[dst_c[sl].at[0]],
                                        add=True)
            def sgrp(g, _):
                pv16 = p_c[sl][0, pl.ds(16 * g, 16)]
                for l in range(16):
                    j = 16 * g + l
                    pj = pv16[l]
                    for u in range(8):
                        rowbuf[sl][j, pl.ds(16 * u, 16)] = (
                            rowbuf[sl][j, pl.ds(16 * u, 16)] * pj)
                return 0
            lax.fori_loop(0, 8, sgrp, 0, unroll=2)
            pltpu.make_async_copy(rowbuf[sl],
                                  acc_sp.at[dst_c[sl].at[0]],
                                  sem_sc[sl]).start(add=True)

        for bi in range(NPB):
            blk = c * NPB + bi
            boffn = blk * n
            den_pass = gat and bi == 0

            if bi > 0:
                def zero_rb(_j, _):
                    for u in range(8):
                        rowbuf[0][_j, pl.ds(16 * u, 16)] = zeros16
                    return 0
                lax.fori_loop(0, 128, zero_rb, 0)
            for off, sz in rslices:
                pltpu.sync_copy(rowbuf[0].at[pl.ds(0, sz)],
                                acc_sp.at[pl.ds(row0 + off, sz)])
            plsc.subcore_barrier()

            idx_load(0, 0)
            compute_srco(0, boffn)
            issue_gathers(0)
            idx_load(nch, 1)
            pltpu.make_async_copy(rowbuf[1],
                                  acc_sp.at[dst_c[1].at[0]],
                                  sem_sc[1]).start(add=True)

            def pair_body(g, _):
                a = 2 * g
                drain_sc(1)
                idx_load(a + 1, 1)
                compute_srco(1, boffn)
                issue_gathers(1)
                process(0, den_pass)
                process(1, den_pass)
                drain_sc(0)
                idx_load(a + 2, 0)
                compute_srco(0, boffn)
                issue_gathers(0)
                return 0
            lax.fori_loop(0, NPAIR, pair_body, 0)

            drain_sc(1)
            wait_gathers(0)
            plsc.subcore_barrier()

            if den_pass:
                for off, sz in rslices:
                    @pl.when(c == 0)
                    def _(off=off, sz=sz):
                        pltpu.sync_copy(den_sp.at[pl.ds(row0 + off, sz)],
                                        bounce_v.at[pl.ds(0, sz)])
                        pltpu.sync_copy(bounce_v.at[pl.ds(0, sz)],
                                        den_ref.at[pl.ds(row0 + off, sz)])
                if with_deg:
                    for off, sz in rslices:
                        @pl.when(c == 1)
                        def _(off=off, sz=sz):
                            pltpu.sync_copy(den_sp.at[pl.ds(row0 + off, sz)],
                                            bounce_v.at[pl.ds(0, sz)])
                            pltpu.sync_copy(bounce_v.at[pl.ds(0, sz)],
                                            deg_ref.at[pl.ds(row0 + off, sz)])
            boffp = blk * NP
            for off, sz in rslices:
                pltpu.sync_copy(acc_sp.at[pl.ds(row0 + off, sz)],
                                rowbuf[1].at[pl.ds(0, sz)])
                pltpu.sync_copy(rowbuf[1].at[pl.ds(0, sz)],
                                acc_ref.at[pl.ds(boffp + row0 + off, sz)])
            plsc.subcore_barrier()

    fn = pl.kernel(body, out_type=out_type, mesh=mesh,
                   scratch_types=scratch,
                   compiler_params=pltpu.CompilerParams(
                       needs_layout_passes=False))
    args = [hb] + list(vals)
    if gat:
        args.append(mx)
    args += [src3, dst3]
    res = fn(*args)
    if not isinstance(res, (tuple, list)):
        res = (res,)
    accf = res[0].reshape(NB, NP, 128)
    den = res[1] if gat else None
    deg = res[2] if with_deg else None
    return accf, den, deg


def _sc_gat(hb, sd, src3, dst3, n, nch, with_deg):
    pself, mx = _softmax_prep(sd)
    asrc = sd[:, 0]
    adst = sd[:, 1]
    NB = hb.shape[0]
    hflat = hb.reshape(NB * n, 128)
    pad = ((0, _JUNK),)
    acc, den, deg = _sc_edge(
        hflat, (jnp.pad(asrc, pad), jnp.pad(adst, pad)), mx,
        src3, dst3, n, "gat_deg" if with_deg else "gat", nch)
    return acc[:, :n, :], den[:n].reshape(n, 1), pself, deg


def kernel(x, edge_index, W1, a_src1, a_dst1, b1, W2, a_src2, a_dst2, b2,
           W3, a_src3, a_dst3, b3, W4, b4):
    n = x.shape[0]
    e = edge_index.shape[1]
    src = edge_index[0].astype(jnp.int32)
    dst = edge_index[1].astype(jnp.int32)

    ept = -(-e // _NT)
    nch = -(-ept // _CK)
    nch += nch % 2
    ncha = nch + 2
    padlen = _NT * nch * _CK - e
    ar = jnp.arange(padlen, dtype=jnp.int32)
    src_p = jnp.concatenate([src, ar % n]).reshape(_NT, nch, _CK)
    dst_p = jnp.concatenate([dst, n + (ar % _JUNK)]).reshape(_NT, nch, _CK)
    arj = jnp.arange(_NT * 2 * _CK, dtype=jnp.int32)
    srcj = (arj % n).reshape(_NT, 2, _CK)
    dstj = (n + (arj % _JUNK)).reshape(_NT, 2, _CK)
    src3 = jnp.concatenate([src_p, srcj], axis=1)
    dst3 = jnp.concatenate([dst_p, dstj], axis=1)

    a2_1 = jnp.stack([a_src1, a_dst1], axis=1)
    a2_2 = jnp.stack([a_src2, a_dst2], axis=1)
    a2_3 = jnp.stack([a_src3, a_dst3], axis=1)

    hb1, sd1 = _proj(x, W1, a2_1)
    acc1, den1, ps1, deg = _sc_gat(hb1, sd1, src3, dst3, n, nch, True)
    hb2, sd2, _ = _combine_proj(acc1, hb1, den1, ps1, b1.reshape(1, -1),
                                W2, a2_2, None)
    acc2, den2, ps2, _ = _sc_gat(hb2, sd2, src3, dst3, n, nch, False)
    hb3, sd3, _ = _combine_proj(acc2, hb2, den2, ps2, b2.reshape(1, -1),
                                W3, a2_3, None)
    acc3, den3, ps3, _ = _sc_gat(hb3, sd3, src3, dst3, n, nch, False)
    hb4, _, dinv = _combine_proj(acc3, hb3, den3, ps3, b3.reshape(1, -1),
                                 W4, None, deg[:n].reshape(n, 1))

    NB4 = hb4.shape[0]
    acc4, _, _ = _sc_edge(hb4.reshape(NB4 * n, 128),
                          (jnp.pad(dinv[:, 0], ((0, _JUNK),)),), None,
                          src3, dst3, n, "gcn", nch)
    z = _combine_gcn(acc4[:, :n, :], hb4, dinv, b4.reshape(1, -1))
    return z
